# baseline (speedup 1.0000x reference)
import jax
import jax.numpy as jnp
from jax import lax
from jax.experimental import pallas as pl
from jax.experimental.pallas import tpu as pltpu
from jax.experimental.pallas import tpu_sc as plsc

B, C, H, W, K = 64, 2, 128, 128, 128
HW = H * W
CHW = C * HW
L = 16
NS = 16
BPW = B // NS


def _sc_body(outflat, maski, ind, tgt, out,
             indals, maskals, tidxs, tpres, idxs, preds,
             partv, redv, outv, shared, sem_is, sem_ms, sem_ts, sem_ps):
    cid = lax.axis_index("c")
    sid = lax.axis_index("s")
    iota = lax.iota(jnp.int32, L)

    @pl.when(cid == 0)
    def _work():
        b0 = sid * BPW
        sti = [pltpu.async_copy(ind.at[b0 + bl], indals[bl], sem_is[bl])
               for bl in range(BPW)]
        stm = [pltpu.async_copy(maski.at[b0 + bl], maskals[bl], sem_ms[bl])
               for bl in range(BPW)]
        tg = []
        for bl in range(BPW):
            tbase = (b0 + bl) * C * K
            for c in range(C):
                r = C * bl + c
                for j in range(8):
                    tidxs[r][pl.ds(L * j, L)] = tbase + (2 * (L * j + iota) + c)
                tg.append(
                    pltpu.async_copy(tgt.at[tidxs[r]], tpres[r], sem_ts[r]))
        pg = []
        for bl in range(BPW):
            sti[bl].wait()
            base = (b0 + bl) * CHW
            for j in range(8):
                sl = pl.ds(L * j, L)
                v = indals[bl][sl] + base
                idxs[2 * bl][sl] = v
                idxs[2 * bl + 1][sl] = v + HW
            for r in (2 * bl, 2 * bl + 1):
                pg.append(
                    pltpu.async_copy(outflat.at[idxs[r]], preds[r], sem_ps[r]))
        for bl in range(BPW):
            stm[bl].wait()
        for cp in tg:
            cp.wait()
        for cp in pg:
            cp.wait()
        acc = jnp.zeros((L,), jnp.float32)
        msum = jnp.zeros((L,), jnp.float32)
        for bl in range(BPW):
            for j in range(8):
                sl = pl.ds(L * j, L)
                mk = maskals[bl][sl].astype(jnp.float32)
                d0 = jnp.abs(preds[2 * bl][sl] - tpres[2 * bl][sl])
                d1 = jnp.abs(preds[2 * bl + 1][sl] - tpres[2 * bl + 1][sl])
                acc = acc + (d0 + d1) * mk
                msum = msum + mk
        partv[pl.ds(0, L)] = acc
        partv[pl.ds(L, L)] = msum
        pltpu.sync_copy(partv, shared.at[sid])
        plsc.subcore_barrier()

        @pl.when(sid == 0)
        def _reduce():
            pltpu.sync_copy(shared, redv)
            ta = jnp.zeros((L,), jnp.float32)
            tm = jnp.zeros((L,), jnp.float32)
            for t in range(NS):
                ta = ta + redv[t, pl.ds(0, L)]
                tm = tm + redv[t, pl.ds(L, L)]
            num = jnp.float32(0.0)
            den = jnp.float32(0.0)
            for i in range(L):
                num = num + ta[i]
                den = den + tm[i]
            den = den * jnp.float32(C) + jnp.float32(1e-4)
            numv = jnp.full((L,), num, jnp.float32)
            denv = jnp.full((L,), den, jnp.float32)
            outv[...] = numv / denv
            pltpu.sync_copy(outv, out)


_SCRATCH = [
    [pltpu.VMEM((K,), jnp.int32) for _ in range(BPW)],
    [pltpu.VMEM((K,), jnp.int32) for _ in range(BPW)],
    [pltpu.VMEM((K,), jnp.int32) for _ in range(2 * BPW)],
    [pltpu.VMEM((K,), jnp.float32) for _ in range(2 * BPW)],
    [pltpu.VMEM((K,), jnp.int32) for _ in range(2 * BPW)],
    [pltpu.VMEM((K,), jnp.float32) for _ in range(2 * BPW)],
    pltpu.VMEM((2 * L,), jnp.float32),
    pltpu.VMEM((NS, 2 * L), jnp.float32),
    pltpu.VMEM((L,), jnp.float32),
    pltpu.VMEM_SHARED((NS, 2 * L), jnp.float32),
    [pltpu.SemaphoreType.DMA for _ in range(BPW)],
    [pltpu.SemaphoreType.DMA for _ in range(BPW)],
    [pltpu.SemaphoreType.DMA for _ in range(2 * BPW)],
    [pltpu.SemaphoreType.DMA for _ in range(2 * BPW)],
]

_sc_call = pl.kernel(
    _sc_body,
    out_type=jax.ShapeDtypeStruct((L,), jnp.float32),
    mesh=plsc.VectorSubcoreMesh(core_axis_name="c", subcore_axis_name="s"),
    scratch_types=_SCRATCH,
)


def kernel(output, mask, ind, target):
    outflat = output.reshape(B * C * H * W)
    mask32 = mask.astype(jnp.int32)
    ind32 = ind.astype(jnp.int32)
    tgt1 = target.reshape(B * K * C)
    res = _sc_call(outflat, mask32, ind32, tgt1)
    return res[0]
